# baseline (speedup 1.0000x reference)
import functools

import jax
import jax.numpy as jnp
from jax import lax
from jax.experimental import pallas as pl
from jax.experimental.pallas import tpu as pltpu
from jax.experimental.pallas import tpu_sc as plsc

_NC = 2
_NS = 16
_NW = _NC * _NS
_CH = 8
_SLOTS = 3
_L = 16
_BS = 512
_S_TC = 1536


def _add_pe_kernel(x_ref, pe_ref, o_ref):
    o_ref[...] = x_ref[...] + pe_ref[...][:, None, :]


def _tc_part(x, pe, s_tc):
    S, B, D = x.shape
    return pl.pallas_call(
        _add_pe_kernel,
        grid=(s_tc // _BS,),
        in_specs=[
            pl.BlockSpec((_BS, B, D), lambda i: (i, 0, 0)),
            pl.BlockSpec((_BS, D), lambda i: (i, 0)),
        ],
        out_specs=pl.BlockSpec((_BS, B, D), lambda i: (i, 0, 0)),
        out_shape=jax.ShapeDtypeStruct((s_tc, B, D), x.dtype),
    )(x, pe)


def _sc_part(x, pe, s_lo):
    S, B, D = x.shape
    n_rows = S - s_lo
    rows_per_w = n_rows // _NW
    n_chunks = rows_per_w // _CH
    dpc = D // _L
    mesh = plsc.VectorSubcoreMesh(core_axis_name="c", subcore_axis_name="s")

    @functools.partial(
        pl.kernel,
        out_type=jax.ShapeDtypeStruct((n_rows, B, D), x.dtype),
        mesh=mesh,
        scratch_types=[
            pltpu.VMEM((_SLOTS, _CH, B, D), jnp.float32),
            pltpu.VMEM((_SLOTS, _CH, D), jnp.float32),
            pltpu.SemaphoreType.DMA((_SLOTS,)),
            pltpu.SemaphoreType.DMA((_SLOTS,)),
        ],
    )
    def sc_add(x_hbm, pe_hbm, out_hbm, xb, pb, sin, sout):
        wid = lax.axis_index("s") * _NC + lax.axis_index("c")
        base = wid * rows_per_w

        in_descs = {}
        out_descs = {}

        def start_in(c):
            slot = c % _SLOTS
            row0 = base + c * _CH
            in_descs[c] = (
                pltpu.async_copy(
                    x_hbm.at[pl.ds(s_lo + row0, _CH)], xb.at[slot],
                    sin.at[slot]),
                pltpu.async_copy(
                    pe_hbm.at[pl.ds(s_lo + row0, _CH)], pb.at[slot],
                    sin.at[slot]),
            )

        start_in(0)
        if n_chunks > 1:
            start_in(1)
        for c in range(n_chunks):
            slot = c % _SLOTS
            dx, dp = in_descs.pop(c)
            dx.wait()
            dp.wait()
            for r in range(_CH):
                @plsc.parallel_loop(0, dpc, unroll=4)
                def _body(dc, _r=r, _slot=slot):
                    sl = pl.ds(dc * _L, _L)
                    pv = pb[_slot, _r, sl]
                    for b in range(B):
                        xb[_slot, _r, b, sl] += pv
            row0 = base + c * _CH
            out_descs[c] = pltpu.async_copy(
                xb.at[slot], out_hbm.at[pl.ds(row0, _CH)], sout.at[slot])
            nxt = c + 2
            if nxt < n_chunks:
                prev = nxt - _SLOTS
                if prev >= 0:
                    out_descs.pop(prev).wait()
                start_in(nxt)
        for c in sorted(out_descs):
            out_descs[c].wait()

    return sc_add(x, pe)


def kernel(x, pos_table):
    S, B, D = x.shape
    pe = pos_table[:S]
    tc_out = _tc_part(x, pe, _S_TC)
    sc_out = _sc_part(x, pe, _S_TC)
    return jnp.concatenate([tc_out, sc_out], axis=0)


# speedup vs baseline: 1.6459x; 1.6459x over previous
import functools

import jax
import jax.numpy as jnp
from jax import lax
from jax.experimental import pallas as pl
from jax.experimental.pallas import tpu as pltpu
from jax.experimental.pallas import tpu_sc as plsc

_NC = 2
_NS = 16
_NW = _NC * _NS
_CH = 4
_SLOTS = 6
_AHEAD = 4
_L = 16


def kernel(x, pos_table):
    S, B, D = x.shape
    pe = pos_table[:S]
    rows_per_w = S // _NW
    n_chunks = rows_per_w // _CH
    dpc = D // _L
    dpc_shift = dpc.bit_length() - 1
    mesh = plsc.VectorSubcoreMesh(core_axis_name="c", subcore_axis_name="s")

    @functools.partial(
        pl.kernel,
        out_type=jax.ShapeDtypeStruct((S, B, D), x.dtype),
        mesh=mesh,
        scratch_types=[
            pltpu.VMEM((_SLOTS, _CH, B, D), jnp.float32),
            pltpu.VMEM((_SLOTS, _CH, D), jnp.float32),
            pltpu.SemaphoreType.DMA((_SLOTS,)),
            pltpu.SemaphoreType.DMA((_SLOTS,)),
        ],
    )
    def sc_add(x_hbm, pe_hbm, out_hbm, xb, pb, sin, sout):
        wid = lax.axis_index("s") * _NC + lax.axis_index("c")
        base = wid * rows_per_w

        in_descs = {}
        out_descs = {}

        def start_in(c):
            slot = c % _SLOTS
            row0 = base + c * _CH
            in_descs[c] = (
                pltpu.async_copy(
                    x_hbm.at[pl.ds(row0, _CH)], xb.at[slot], sin.at[slot]),
                pltpu.async_copy(
                    pe_hbm.at[pl.ds(row0, _CH)], pb.at[slot], sin.at[slot]),
            )

        for c in range(min(_AHEAD, n_chunks)):
            start_in(c)
        for c in range(n_chunks):
            slot = c % _SLOTS
            dx, dp = in_descs.pop(c)
            dx.wait()
            dp.wait()

            @plsc.parallel_loop(0, _CH * dpc, unroll=4)
            def _body(i, _slot=slot):
                r = lax.shift_right_logical(i, dpc_shift)
                dc = lax.bitwise_and(i, dpc - 1)
                sl = pl.ds(dc * _L, _L)
                pv = pb[_slot, r, sl]
                for b in range(B):
                    xb[_slot, r, b, sl] += pv

            row0 = base + c * _CH
            out_descs[c] = pltpu.async_copy(
                xb.at[slot], out_hbm.at[pl.ds(row0, _CH)], sout.at[slot])
            nxt = c + _AHEAD
            if nxt < n_chunks:
                prev = nxt - _SLOTS
                if prev >= 0:
                    out_descs.pop(prev).wait()
                start_in(nxt)
        for c in sorted(out_descs):
            out_descs[c].wait()

    return sc_add(x, pe)
